# baseline (speedup 1.0000x reference)
import functools

import jax
import jax.numpy as jnp
from jax import lax
from jax.experimental import pallas as pl
from jax.experimental.pallas import tpu as pltpu
from jax.experimental.pallas import tpu_sc as plsc

NC = 2
NS = 16
NW = NC * NS
DP = 128
CHUNK = 128
GROUP = 2


@functools.partial(jax.jit, static_argnames=("total",))
def _sc_gather(idx_e, idx_o, table, total):
    d = table.shape[1]
    half = total // 2
    bpw = half // NW
    nchunk = bpw // CHUNK
    ngroup = nchunk // GROUP
    gr = GROUP * CHUNK

    mesh = plsc.VectorSubcoreMesh(core_axis_name="c", subcore_axis_name="s")

    @functools.partial(
        pl.kernel,
        out_type=jax.ShapeDtypeStruct((half, DP), jnp.float32),
        mesh=mesh,
        scratch_types=[
            pltpu.VMEM((nchunk, CHUNK), jnp.int32),
            pltpu.VMEM((nchunk, CHUNK), jnp.int32),
            pltpu.VMEM((2, 2, gr, d), jnp.float32),
            pltpu.SemaphoreType.DMA,
            pltpu.SemaphoreType.DMA,
            pltpu.SemaphoreType.DMA,
            pltpu.SemaphoreType.DMA,
        ],
        compiler_params=pltpu.CompilerParams(
            use_tc_tiling_on_sc=False,
            skip_device_barrier=True,
            disable_bounds_checks=True,
            disable_semaphore_checks=True,
        ),
    )
    def gather_kernel(idx_e_hbm, idx_o_hbm, table_hbm, out_hbm,
                      idx_ev, idx_ov, rows_v,
                      gsem0, gsem1, osem0, osem1):
        wid = lax.axis_index("s") * NC + lax.axis_index("c")
        base = wid * bpw
        gsem = (gsem0, gsem1)
        osem = (osem0, osem1)
        pltpu.sync_copy(idx_e_hbm.at[pl.ds(wid * nchunk, nchunk)], idx_ev)
        pltpu.sync_copy(idx_o_hbm.at[pl.ds(wid * nchunk, nchunk)], idx_ov)

        def issue_gathers(g, s):
            for u in range(GROUP):
                c = g * GROUP + u
                r0 = u * CHUNK
                for p, iv in ((0, idx_ev), (1, idx_ov)):
                    pltpu.async_copy(
                        table_hbm.at[iv.at[c]],
                        rows_v.at[s, p, pl.ds(r0, CHUNK)],
                        gsem[s],
                    )

        def drain_gathers(s):
            for u in range(2 * GROUP):
                pltpu.make_async_copy(
                    table_hbm.at[idx_ev.at[0]],
                    rows_v.at[s, 0, pl.ds(0, CHUNK)],
                    gsem[s],
                ).wait()

        def out_copy(g, s, issue):
            for p in range(2):
                src = rows_v.at[s, p]
                dst = out_hbm.at[pl.ds(base + g * gr, gr), pl.ds(p * d, d)]
                if issue:
                    pltpu.async_copy(src, dst, osem[s])
                else:
                    pltpu.make_async_copy(src, dst, osem[s]).wait()

        issue_gathers(0, 0)

        @pl.loop(0, ngroup, step=2)
        def _(go):
            for s in range(2):
                g = go + s
                s2 = 1 - s
                drain_gathers(s)
                out_copy(g, s, issue=True)
                if s == 0:
                    @pl.when(go > 0)
                    def _():
                        out_copy(0, s2, issue=False)
                    issue_gathers(g + 1, s2)
                else:
                    out_copy(0, s2, issue=False)

                    @pl.when(go + 2 < ngroup)
                    def _():
                        issue_gathers(g + 1, s2)

        out_copy(0, 1, issue=False)

    return gather_kernel(idx_e, idx_o, table)


def _to_native_kernel(h, d, x_ref, o_ref):
    x = x_ref[...]
    bc = 2 * x.shape[0] // h
    x3 = x.reshape(bc, h // 2, DP)
    for t in range(h):
        o_ref[t] = x3[:, t // 2, (t % 2) * d:(t % 2) * d + d].T


@functools.partial(jax.jit, static_argnames=("b", "h", "d"))
def _to_native(rows, b, h, d):
    bc = 256
    return pl.pallas_call(
        functools.partial(_to_native_kernel, h, d),
        grid=(b // bc,),
        in_specs=[pl.BlockSpec((bc * h // 2, DP), lambda i: (i, 0))],
        out_specs=pl.BlockSpec((h, d, bc), lambda i: (0, 0, i)),
        out_shape=jax.ShapeDtypeStruct((h, d, b), jnp.float32),
    )(rows)


def kernel(category_ids, embedding_weight):
    b, h = category_ids.shape
    total = b * h
    d = embedding_weight.shape[1]
    idx_e = category_ids[:, 0::2].reshape(total // 2 // CHUNK, CHUNK)
    idx_o = category_ids[:, 1::2].reshape(total // 2 // CHUNK, CHUNK)
    idx_e = idx_e.astype(jnp.int32)
    idx_o = idx_o.astype(jnp.int32)
    rows = _sc_gather(idx_e, idx_o, embedding_weight, total)
    out_native = _to_native(rows, b, h, d)
    return out_native.transpose(2, 0, 1)


# speedup vs baseline: 1.0029x; 1.0029x over previous
import functools

import jax
import jax.numpy as jnp
from jax import lax
from jax.experimental import pallas as pl
from jax.experimental.pallas import tpu as pltpu
from jax.experimental.pallas import tpu_sc as plsc

NC = 2
NS = 16
NW = NC * NS
DP = 128
CHUNK = 128
GROUP = 2


@functools.partial(jax.jit, static_argnames=("total",))
def _sc_gather(idx_e, idx_o, table, total):
    d = table.shape[1]
    half = total // 2
    bpw = half // NW
    nchunk = bpw // CHUNK
    ngroup = nchunk // GROUP
    gr = GROUP * CHUNK

    mesh = plsc.VectorSubcoreMesh(core_axis_name="c", subcore_axis_name="s")

    @functools.partial(
        pl.kernel,
        out_type=jax.ShapeDtypeStruct((half, DP), jnp.float32),
        mesh=mesh,
        scratch_types=[
            pltpu.VMEM((nchunk, CHUNK), jnp.int32),
            pltpu.VMEM((nchunk, CHUNK), jnp.int32),
            pltpu.VMEM((2, 2, gr, d), jnp.float32),
            pltpu.SemaphoreType.DMA,
            pltpu.SemaphoreType.DMA,
            pltpu.SemaphoreType.DMA,
            pltpu.SemaphoreType.DMA,
        ],
        compiler_params=pltpu.CompilerParams(
            use_tc_tiling_on_sc=False,
            skip_device_barrier=True,
            disable_bounds_checks=True,
            disable_semaphore_checks=True,
        ),
    )
    def gather_kernel(idx_e_hbm, idx_o_hbm, table_hbm, out_hbm,
                      idx_ev, idx_ov, rows_v,
                      gsem0, gsem1, osem0, osem1):
        wid = lax.axis_index("s") * NC + lax.axis_index("c")
        base = wid * bpw
        gsem = (gsem0, gsem1)
        osem = (osem0, osem1)
        pltpu.sync_copy(idx_e_hbm.at[pl.ds(wid * nchunk, nchunk)], idx_ev)
        pltpu.sync_copy(idx_o_hbm.at[pl.ds(wid * nchunk, nchunk)], idx_ov)

        def issue_gathers(g, s):
            for u in range(GROUP):
                c = g * GROUP + u
                r0 = u * CHUNK
                for p, iv in ((0, idx_ev), (1, idx_ov)):
                    pltpu.async_copy(
                        table_hbm.at[iv.at[c]],
                        rows_v.at[s, p, pl.ds(r0, CHUNK)],
                        gsem[s],
                    )

        def drain_gathers(s):
            for u in range(2 * GROUP):
                pltpu.make_async_copy(
                    table_hbm.at[idx_ev.at[0]],
                    rows_v.at[s, 0, pl.ds(0, CHUNK)],
                    gsem[s],
                ).wait()

        def out_copy(g, s, issue):
            for p in range(2):
                src = rows_v.at[s, p]
                dst = out_hbm.at[pl.ds(base + g * gr, gr), pl.ds(p * d, d)]
                if issue:
                    pltpu.async_copy(src, dst, osem[s])
                else:
                    pltpu.make_async_copy(src, dst, osem[s]).wait()

        issue_gathers(0, 0)

        @pl.loop(0, ngroup, step=2)
        def _(go):
            for s in range(2):
                g = go + s
                s2 = 1 - s
                drain_gathers(s)
                out_copy(g, s, issue=True)
                if s == 0:
                    @pl.when(go > 0)
                    def _():
                        out_copy(0, s2, issue=False)
                    issue_gathers(g + 1, s2)
                else:
                    out_copy(0, s2, issue=False)

                    @pl.when(go + 2 < ngroup)
                    def _():
                        issue_gathers(g + 1, s2)

        out_copy(0, 1, issue=False)

    return gather_kernel(idx_e, idx_o, table)


def _to_native_kernel(h, d, x_ref, o_ref):
    x = x_ref[...]
    bc = 2 * x.shape[0] // h
    x3 = x.reshape(bc, h // 2, DP)
    for t in range(h):
        o_ref[t] = x3[:, t // 2, (t % 2) * d:(t % 2) * d + d].T


@functools.partial(jax.jit, static_argnames=("b", "h", "d"))
def _to_native(rows, b, h, d):
    bc = 128
    return pl.pallas_call(
        functools.partial(_to_native_kernel, h, d),
        grid=(b // bc,),
        in_specs=[pl.BlockSpec((bc * h // 2, DP), lambda i: (i, 0))],
        out_specs=pl.BlockSpec((h, d, bc), lambda i: (0, 0, i)),
        out_shape=jax.ShapeDtypeStruct((h, d, b), jnp.float32),
    )(rows)


def kernel(category_ids, embedding_weight):
    b, h = category_ids.shape
    total = b * h
    d = embedding_weight.shape[1]
    idx_e = category_ids[:, 0::2].reshape(total // 2 // CHUNK, CHUNK)
    idx_o = category_ids[:, 1::2].reshape(total // 2 // CHUNK, CHUNK)
    idx_e = idx_e.astype(jnp.int32)
    idx_o = idx_o.astype(jnp.int32)
    rows = _sc_gather(idx_e, idx_o, embedding_weight, total)
    out_native = _to_native(rows, b, h, d)
    return out_native.transpose(2, 0, 1)
